# baseline (speedup 1.0000x reference)
import functools

import jax
import jax.numpy as jnp
from jax import lax
from jax.experimental import pallas as pl
from jax.experimental.pallas import tpu as pltpu
from jax.experimental.pallas import tpu_sc as plsc

USER = 4000
ITEM = 6000
N = USER + ITEM
E = 320000
D = 128
NC = 2
NS = 16
NW = NC * NS
CH = 128
NCH = 80
EPAD = NW * NCH * CH
NPAD = 10240
RPT = NPAD // NS


def _lane_bcast(v, r):
    idx = jnp.full((16, 1), r, dtype=jnp.int32)
    dn = lax.GatherDimensionNumbers(
        offset_dims=(), collapsed_slice_dims=(0,), start_index_map=(0,))
    return lax.gather(v, idx, dn, (1,),
                      mode=lax.GatherScatterMode.PROMISE_IN_BOUNDS)


_mesh = plsc.VectorSubcoreMesh(core_axis_name="c", subcore_axis_name="s")


@functools.partial(
    pl.kernel,
    mesh=_mesh,
    out_type=jax.ShapeDtypeStruct((NC, NPAD, D), jnp.float32),
    scratch_types=[
        pltpu.VMEM((NCH, CH), jnp.int32),
        pltpu.VMEM((NCH, CH), jnp.int32),
        pltpu.VMEM((NCH, CH), jnp.float32),
        pltpu.VMEM((CH, D), jnp.float32),
        pltpu.VMEM_SHARED((NPAD, D), jnp.float32),
        pltpu.SemaphoreType.DMA,
    ],
)
def _spmm(table_hbm, rows_hbm, cols_hbm, vals_hbm, zeros_hbm, out_hbm,
          rows_v, cols_v, vals_v, gbuf, acc, sem):
    c = lax.axis_index("c")
    s = lax.axis_index("s")
    wid = s * NC + c

    pltpu.sync_copy(zeros_hbm.at[pl.ds(s * RPT, RPT)],
                    acc.at[pl.ds(s * RPT, RPT)])

    pltpu.sync_copy(rows_hbm.at[wid], rows_v)
    pltpu.sync_copy(cols_hbm.at[wid], cols_v)
    pltpu.sync_copy(vals_hbm.at[wid], vals_v)
    plsc.subcore_barrier()

    def body(g, carry):
        pltpu.async_copy(table_hbm.at[cols_v.at[g]], gbuf, sem).wait()

        def scale(sub, carry2):
            vv = vals_v[g, pl.ds(sub * 16, 16)]
            for rr in range(16):
                r = sub * 16 + rr
                vs = _lane_bcast(vv, rr)
                for d in range(D // 16):
                    sl = pl.ds(d * 16, 16)
                    gbuf[r, sl] = gbuf[r, sl] * vs
            return carry2

        lax.fori_loop(0, CH // 16, scale, 0)
        pltpu.sync_copy(gbuf, acc.at[rows_v.at[g]], add=True)
        return carry

    lax.fori_loop(0, NCH, body, 0)
    plsc.subcore_barrier()

    pltpu.sync_copy(acc.at[pl.ds(s * RPT, RPT)],
                    out_hbm.at[c, pl.ds(s * RPT, RPT)])


def _comb_body(p_ref, acc_ref, t_ref, accout_ref):
    t = p_ref[0] + p_ref[1]
    t_ref[...] = t
    accout_ref[...] = acc_ref[...] + t


_BR = 1024

_combine = pl.pallas_call(
    _comb_body,
    grid=(NPAD // _BR,),
    in_specs=[
        pl.BlockSpec((NC, _BR, D), lambda i: (0, i, 0)),
        pl.BlockSpec((_BR, D), lambda i: (i, 0)),
    ],
    out_specs=[
        pl.BlockSpec((_BR, D), lambda i: (i, 0)),
        pl.BlockSpec((_BR, D), lambda i: (i, 0)),
    ],
    out_shape=[
        jax.ShapeDtypeStruct((NPAD, D), jnp.float32),
        jax.ShapeDtypeStruct((NPAD, D), jnp.float32),
    ],
)


def kernel(adj_indices, adj_values, user_embedding, item_embedding):
    pad = EPAD - E
    rows = jnp.pad(adj_indices[0].astype(jnp.int32), (0, pad))
    cols = jnp.pad(adj_indices[1].astype(jnp.int32), (0, pad))
    vals = jnp.pad(adj_values, (0, pad))

    wpe = EPAD // NW
    eid = jnp.arange(EPAD, dtype=jnp.int32)
    key = ((eid // wpe) << 14) | cols
    order = jnp.argsort(key)
    rows = rows[order].reshape(NW, NCH, CH)
    cols = cols[order].reshape(NW, NCH, CH)
    vals = vals[order].reshape(NW, NCH, CH)

    e0 = jnp.concatenate([user_embedding, item_embedding], axis=0)
    e0p = jnp.pad(e0, ((0, NPAD - N), (0, 0)))
    zeros = jnp.zeros((NPAD, D), jnp.float32)

    table = e0p
    acc = e0p
    for _ in range(3):
        partials = _spmm(table, rows, cols, vals, zeros)
        table, acc = _combine(partials, acc)

    out = acc[:N]
    return (out, out[:USER], out[USER:])


# speedup vs baseline: 1.5637x; 1.5637x over previous
import functools

import jax
import jax.numpy as jnp
from jax import lax
from jax.experimental import pallas as pl
from jax.experimental.pallas import tpu as pltpu
from jax.experimental.pallas import tpu_sc as plsc

USER = 4000
ITEM = 6000
N = USER + ITEM
E = 320000
D = 128
NC = 2
NS = 16
NW = NC * NS
CH = 64
NCH = 160
WPE = NCH * CH
EPAD = NW * WPE
NPAD = 10240
RPT = NPAD // NS


def _lane_bcast(v, r):
    idx = jnp.full((16, 1), r, dtype=jnp.int32)
    dn = lax.GatherDimensionNumbers(
        offset_dims=(), collapsed_slice_dims=(0,), start_index_map=(0,))
    return lax.gather(v, idx, dn, (1,),
                      mode=lax.GatherScatterMode.PROMISE_IN_BOUNDS)


_mesh = plsc.VectorSubcoreMesh(core_axis_name="c", subcore_axis_name="s")


@functools.partial(
    pl.kernel,
    mesh=_mesh,
    out_type=jax.ShapeDtypeStruct((NC, NPAD, D), jnp.float32),
    scratch_types=[
        pltpu.VMEM((WPE,), jnp.int32),
        pltpu.VMEM((WPE,), jnp.int32),
        pltpu.VMEM((WPE,), jnp.float32),
        pltpu.VMEM((2, CH), jnp.int32),
        pltpu.VMEM((CH, D), jnp.float32),
        pltpu.VMEM((CH, D), jnp.float32),
        pltpu.VMEM_SHARED((NPAD, D), jnp.float32),
        pltpu.SemaphoreType.DMA,
        pltpu.SemaphoreType.DMA,
    ],
)
def _spmm(table_hbm, rows_hbm, cols_hbm, vals_hbm, zeros_hbm, out_hbm,
          rows_v, cols_v, vals_v, rowring, gb0, gb1, acc, sg0, sg1):
    gb = (gb0, gb1)
    sg = (sg0, sg1)
    c = lax.axis_index("c")
    s = lax.axis_index("s")
    wid = s * NC + c

    pltpu.sync_copy(zeros_hbm.at[pl.ds(s * RPT, RPT)],
                    acc.at[pl.ds(s * RPT, RPT)])

    base = wid * WPE
    pltpu.sync_copy(rows_hbm.at[pl.ds(base, WPE)], rows_v)
    pltpu.sync_copy(cols_hbm.at[pl.ds(base, WPE)], cols_v)
    pltpu.sync_copy(vals_hbm.at[pl.ds(base, WPE)], vals_v)
    plsc.subcore_barrier()

    pltpu.async_copy(table_hbm.at[cols_v.at[pl.ds(0, CH)]], gb[0], sg[0])

    def body(p, carry):
        for b in range(2):
            g = p * 2 + b
            gp1 = jnp.minimum(g + 1, NCH - 1)
            pltpu.async_copy(
                table_hbm.at[cols_v.at[pl.ds(gp1 * CH, CH)]],
                gb[1 - b], sg[1 - b])
            pltpu.make_async_copy(
                table_hbm.at[cols_v.at[pl.ds(0, CH)]], gb[b],
                sg[b]).wait()

            off = g * CH
            for t in range(CH // 16):
                vv = vals_v[pl.ds(off + 16 * t, 16)]
                for rr in range(16):
                    r = 16 * t + rr
                    vs = _lane_bcast(vv, rr)
                    for d in range(D // 16):
                        sl = pl.ds(d * 16, 16)
                        gb[b][r, sl] = gb[b][r, sl] * vs
                rowring[b, pl.ds(16 * t, 16)] = (
                    rows_v[pl.ds(off + 16 * t, 16)])

            pltpu.sync_copy(gb[b], acc.at[rowring.at[b]], add=True)
        return carry

    lax.fori_loop(0, NCH // 2, body, 0)

    pltpu.make_async_copy(table_hbm.at[cols_v.at[pl.ds(0, CH)]], gb[0],
                          sg[0]).wait()
    plsc.subcore_barrier()

    pltpu.sync_copy(acc.at[pl.ds(s * RPT, RPT)],
                    out_hbm.at[c, pl.ds(s * RPT, RPT)])


def _comb_body(p_ref, acc_ref, t_ref, accout_ref):
    t = p_ref[0] + p_ref[1]
    t_ref[...] = t
    accout_ref[...] = acc_ref[...] + t


_BR = 1024

_combine = pl.pallas_call(
    _comb_body,
    grid=(NPAD // _BR,),
    in_specs=[
        pl.BlockSpec((NC, _BR, D), lambda i: (0, i, 0)),
        pl.BlockSpec((_BR, D), lambda i: (i, 0)),
    ],
    out_specs=[
        pl.BlockSpec((_BR, D), lambda i: (i, 0)),
        pl.BlockSpec((_BR, D), lambda i: (i, 0)),
    ],
    out_shape=[
        jax.ShapeDtypeStruct((NPAD, D), jnp.float32),
        jax.ShapeDtypeStruct((NPAD, D), jnp.float32),
    ],
)


def kernel(adj_indices, adj_values, user_embedding, item_embedding):
    pad = EPAD - E
    rows = jnp.pad(adj_indices[0].astype(jnp.int32), (0, pad))
    cols = jnp.pad(adj_indices[1].astype(jnp.int32), (0, pad))
    vals = jnp.pad(adj_values, (0, pad))

    e0 = jnp.concatenate([user_embedding, item_embedding], axis=0)
    e0p = jnp.pad(e0, ((0, NPAD - N), (0, 0)))
    zeros = jnp.zeros((NPAD, D), jnp.float32)

    table = e0p
    acc = e0p
    for _ in range(3):
        partials = _spmm(table, rows, cols, vals, zeros)
        table, acc = _combine(partials, acc)

    out = acc[:N]
    return (out, out[:USER], out[USER:])
